# Initial kernel scaffold; baseline (speedup 1.0000x reference)
import jax
import jax.numpy as jnp
from jax.experimental import pallas as pl


def kernel(x, edge_index, mask1, W1, b1, mask2, W2, b2):
    raise NotImplementedError("write your pallas kernel here")


# speedup vs baseline: 18.0644x; 18.0644x over previous
import functools

import jax
import jax.numpy as jnp
from jax import lax
from jax.experimental import pallas as pl
from jax.experimental.pallas import tpu as pltpu
from jax.experimental.pallas import tpu_sc as plsc

N = 10000
N_PAD = 10240
E = 320000
D_IN = 128
D_HID = 128
D_OUT = 64

B = 125
GROUPS = E // B
NC = 2
NS = 16
NW = NC * NS
GPW = GROUPS // NW
RPT = N_PAD // NS
CHUNK = 128
LANES = 16


def _fill2d(ref, nrows, ncols, value):
    v = jnp.full((LANES,), value, jnp.float32)

    def row(i, _):
        def col(j, _):
            ref[i, pl.ds(j * LANES, LANES)] = v
            return 0

        return lax.fori_loop(0, ncols // LANES, col, 0)

    lax.fori_loop(0, nrows, row, 0)


def _sc_mesh():
    return plsc.VectorSubcoreMesh(core_axis_name="c", subcore_axis_name="s")


_SC_PARAMS = pltpu.CompilerParams(use_tc_tiling_on_sc=False)


def _make_deg_kernel():
    @functools.partial(
        pl.kernel,
        out_type=jax.ShapeDtypeStruct((NC, N_PAD, LANES), jnp.float32),
        mesh=_sc_mesh(),
        compiler_params=_SC_PARAMS,
        scratch_types=[
            pltpu.VMEM((GPW, B), jnp.int32),
            pltpu.VMEM((B, LANES), jnp.float32),
            pltpu.VMEM((CHUNK, LANES), jnp.float32),
            pltpu.VMEM_SHARED((N_PAD, LANES), jnp.float32),
        ],
    )
    def deg_kernel(dst_hbm, out_hbm, dst_v, ones_v, zw_v, acc):
        c = lax.axis_index("c")
        s = lax.axis_index("s")
        _fill2d(zw_v, CHUNK, LANES, 0.0)
        r0 = s * RPT
        for k in range(RPT // CHUNK):
            pltpu.sync_copy(zw_v, acc.at[pl.ds(r0 + k * CHUNK, CHUNK)])
        g0 = (c * NS + s) * GPW
        pltpu.sync_copy(dst_hbm.at[pl.ds(g0, GPW)], dst_v)
        _fill2d(ones_v, B, LANES, 1.0)
        plsc.subcore_barrier()

        def body(j, _):
            pltpu.sync_copy(ones_v, acc.at[dst_v.at[j]], add=True)
            return 0

        lax.fori_loop(0, GPW, body, 0)
        plsc.subcore_barrier()
        for k in range(RPT // CHUNK):
            pltpu.sync_copy(acc.at[pl.ds(r0 + k * CHUNK, CHUNK)], zw_v)
            pltpu.sync_copy(zw_v, out_hbm.at[c, pl.ds(r0 + k * CHUNK, CHUNK)])

    return deg_kernel


def _make_agg_kernel(D):
    @functools.partial(
        pl.kernel,
        out_type=jax.ShapeDtypeStruct((NC, N_PAD, D), jnp.float32),
        mesh=_sc_mesh(),
        compiler_params=_SC_PARAMS,
        scratch_types=[
            pltpu.VMEM((GPW, B), jnp.int32),
            pltpu.VMEM((GPW, B), jnp.int32),
            pltpu.VMEM((B, D), jnp.float32),
            pltpu.VMEM((CHUNK, D), jnp.float32),
            pltpu.VMEM_SHARED((N_PAD, D), jnp.float32),
            pltpu.SemaphoreType.DMA,
        ],
    )
    def agg_kernel(g_hbm, src_hbm, dst_hbm, out_hbm, src_v, dst_v, rows_v, zw_v, acc, sem):
        c = lax.axis_index("c")
        s = lax.axis_index("s")
        _fill2d(zw_v, CHUNK, D, 0.0)
        r0 = s * RPT
        for k in range(RPT // CHUNK):
            pltpu.sync_copy(zw_v, acc.at[pl.ds(r0 + k * CHUNK, CHUNK)])
        g0 = (c * NS + s) * GPW
        pltpu.sync_copy(src_hbm.at[pl.ds(g0, GPW)], src_v)
        pltpu.sync_copy(dst_hbm.at[pl.ds(g0, GPW)], dst_v)
        plsc.subcore_barrier()

        def body(j, _):
            pltpu.async_copy(g_hbm.at[src_v.at[j]], rows_v, sem).wait()
            pltpu.sync_copy(rows_v, acc.at[dst_v.at[j]], add=True)
            return 0

        lax.fori_loop(0, GPW, body, 0)
        plsc.subcore_barrier()
        for k in range(RPT // CHUNK):
            pltpu.sync_copy(acc.at[pl.ds(r0 + k * CHUNK, CHUNK)], zw_v)
            pltpu.sync_copy(zw_v, out_hbm.at[c, pl.ds(r0 + k * CHUNK, CHUNK)])

    return agg_kernel


_deg = _make_deg_kernel()
_agg64 = _make_agg_kernel(64)


def _tc_matmul1(x, mask1, W1):
    def body(x_ref, m_ref, w_ref, o_ref):
        o_ref[...] = jnp.dot(x_ref[...] * m_ref[...], w_ref[...],
                             preferred_element_type=jnp.float32)

    return pl.pallas_call(
        body, out_shape=jax.ShapeDtypeStruct((N, D_HID), jnp.float32)
    )(x, mask1, W1)


def _tc_scale(h, dega, degb):
    def body(h_ref, da_ref, db_ref, g_ref, dinv_ref):
        dinv = lax.rsqrt(da_ref[...] + db_ref[...] + 1.0)
        dinv_ref[...] = dinv
        g_ref[...] = h_ref[...] * dinv

    return pl.pallas_call(
        body,
        out_shape=[
            jax.ShapeDtypeStruct((N, D_HID), jnp.float32),
            jax.ShapeDtypeStruct((N, 1), jnp.float32),
        ],
    )(h, dega, degb)


def _tc_mid(sa0, sb0, sa1, sb1, g1, dinv, b1, mask2, W2):
    def body(sa0_ref, sb0_ref, sa1_ref, sb1_ref, g1_ref, dinv_ref, b1_ref,
             m2_ref, w2_ref, o_ref):
        s1 = jnp.concatenate(
            [sa0_ref[...] + sb0_ref[...], sa1_ref[...] + sb1_ref[...]], axis=1)
        u = (s1 + g1_ref[...]) * dinv_ref[...] + b1_ref[...]
        u = jnp.maximum(u, 0.0)
        o_ref[...] = jnp.dot(u * m2_ref[...], w2_ref[...],
                             preferred_element_type=jnp.float32) * dinv_ref[...]

    return pl.pallas_call(
        body, out_shape=jax.ShapeDtypeStruct((N, D_OUT), jnp.float32)
    )(sa0, sb0, sa1, sb1, g1, dinv, b1, mask2, W2)


def _tc_final(sa, sb, g2, dinv, b2):
    def body(sa_ref, sb_ref, g2_ref, dinv_ref, b2_ref, o_ref):
        o_ref[...] = (sa_ref[...] + sb_ref[...] + g2_ref[...]) * dinv_ref[...] + b2_ref[...]

    return pl.pallas_call(
        body, out_shape=jax.ShapeDtypeStruct((N, D_OUT), jnp.float32)
    )(sa, sb, g2, dinv, b2)


def kernel(x, edge_index, mask1, W1, b1, mask2, W2, b2):
    src2 = edge_index[0].reshape(GROUPS, B)
    dst2 = edge_index[1].reshape(GROUPS, B)

    degp = _deg(dst2)
    h1 = _tc_matmul1(x, mask1.reshape(1, D_IN), W1)
    dega = degp[0, :N, 0:1]
    degb = degp[1, :N, 0:1]
    g1, dinv = _tc_scale(h1, dega, degb)

    S1l = _agg64(g1[:, :64], src2, dst2)
    S1r = _agg64(g1[:, 64:], src2, dst2)
    g2 = _tc_mid(S1l[0, :N], S1l[1, :N], S1r[0, :N], S1r[1, :N],
                 g1, dinv, b1.reshape(1, D_HID), mask2.reshape(1, D_HID), W2)

    S2 = _agg64(g2, src2, dst2)
    return _tc_final(S2[0, :N], S2[1, :N], g2, dinv, b2.reshape(1, D_OUT))
